# baseline (speedup 1.0000x reference)
import functools

import jax
import jax.numpy as jnp
from jax.experimental import pallas as pl
from jax.experimental.pallas import tpu as pltpu

_NCHAINS = 4


def _count_ge(mask):
    m = jnp.where(mask, jnp.bfloat16(1), jnp.bfloat16(0))
    n = m.shape[-1]
    while n > 128:
        n //= 2
        m = m[:, :n] + m[:, n:]
    ones = jnp.ones((128, 128), jnp.bfloat16)
    cnt = jax.lax.dot_general(m, ones, (((1,), (0,)), ((), ())),
                              preferred_element_type=jnp.float32)
    return cnt[:, :1]


def _chains(a):
    h = a.shape[0] // _NCHAINS
    return tuple(a[i * h:(i + 1) * h] for i in range(_NCHAINS))


def _topk_mask_kernel(x_ref, mask_ref, o_ref, *, k: int):
    x = x_ref[0, 0] + mask_ref[0, 0]
    rows, n = x.shape
    kf = jnp.float32(k)

    y = jax.lax.bitcast_convert_type(x, jnp.int32)
    key = jnp.where(y < 0, y ^ jnp.int32(0x7FFFFFFF), y)

    hi16 = jax.lax.shift_right_arithmetic(key, 16).astype(jnp.int16)
    lo_s = (key ^ jnp.int32(0x8000)).astype(jnp.int16)

    hi_c = _chains(hi16)
    z = tuple(jnp.zeros((rows // _NCHAINS, 1), jnp.int32)
              for _ in range(_NCHAINS))
    nf = tuple(jnp.full((rows // _NCHAINS, 1), n, jnp.float32)
               for _ in range(_NCHAINS))

    def hi_step(i, st):
        p, g = st
        j = 15 - i
        bit = jax.lax.shift_left(jnp.int32(1), j)
        c = tuple(pc | bit for pc in p)
        cnt = tuple(_count_ge(hc >= (cc - 32768).astype(jnp.int16))
                    for hc, cc in zip(hi_c, c))
        keep = tuple(cc >= kf for cc in cnt)
        return (tuple(jnp.where(kp, cc, pc)
                      for kp, cc, pc in zip(keep, c, p)),
                tuple(jnp.where(kp, cc, gc)
                      for kp, cc, gc in zip(keep, cnt, g)))

    p_hi_c, cge_c = jax.lax.fori_loop(0, 16, hi_step, (z, nf))
    p_hi = jnp.concatenate(p_hi_c, axis=0)
    thr_hi = (p_hi - 32768).astype(jnp.int16)

    hi_eq = hi16 == thr_hi
    cnt_gt_hi = _count_ge(hi16 > thr_hi)
    k_b = kf - cnt_gt_hi
    cnt_eq_hi = jnp.concatenate(cge_c, axis=0) - cnt_gt_hi

    lo16 = jnp.where(hi_eq, lo_s, jnp.int16(-32768))
    lo_c = _chains(lo16)
    kb_c = _chains(k_b)
    ceq_c = _chains(cnt_eq_hi)

    def lo_step_factory(jbase):
        def lo_step(i, st):
            p, g = st
            j = jbase - i
            bit = jax.lax.shift_left(jnp.int32(1), j)
            c = tuple(pc | bit for pc in p)
            cnt = tuple(_count_ge(lc >= (cc - 32768).astype(jnp.int16))
                        for lc, cc in zip(lo_c, c))
            keep = tuple(cc >= kc for cc, kc in zip(cnt, kb_c))
            return (tuple(jnp.where(kp, cc, pc)
                          for kp, cc, pc in zip(keep, c, p)),
                    tuple(jnp.where(kp, cc, gc)
                          for kp, cc, gc in zip(keep, cnt, g)))
        return lo_step

    st8 = jax.lax.fori_loop(0, 8, lo_step_factory(15), (z, ceq_c))

    def run_lo8():
        return jax.lax.fori_loop(0, 8, lo_step_factory(7), st8)

    conf_c = [jnp.any(gc != kc) for gc, kc in zip(st8[1], kb_c)]
    lo_conflict = conf_c[0]
    for cc in conf_c[1:]:
        lo_conflict = lo_conflict | cc
    p_lo_c, cbf_c = jax.lax.cond(lo_conflict, run_lo8, lambda: st8)
    p_lo = jnp.concatenate(p_lo_c, axis=0)
    thr_lo = (p_lo - 32768).astype(jnp.int16)

    gt = (hi16 > thr_hi) | (lo16 > thr_lo)
    eq = hi_eq & (lo16 == thr_lo)

    cnt_ge_tot = cnt_gt_hi + jnp.concatenate(cbf_c, axis=0)
    tie_conflict = jnp.any(cnt_ge_tot != kf)

    idx16 = jax.lax.broadcasted_iota(jnp.int32, (rows, n), 1).astype(
        jnp.int16)

    def run_tie():
        cnt_gt = _count_ge(gt)
        need = kf - cnt_gt
        nd_c = _chains(need)
        idxm = jnp.where(eq, idx16, jnp.int16(32767))
        ix_c = _chains(idxm)

        def idx_step(i, m):
            j = 10 - i
            bit = jax.lax.shift_left(jnp.int32(1), j)
            c = tuple(mc | (bit - 1) for mc in m)
            cnt = tuple(_count_ge(ic <= cc.astype(jnp.int16))
                        for ic, cc in zip(ix_c, c))
            return tuple(jnp.where(cc >= nc, mc, mc | bit)
                         for cc, nc, mc in zip(cnt, nd_c, m))

        return jax.lax.fori_loop(0, 11, idx_step, z)

    nm1 = tuple(jnp.full((rows // _NCHAINS, 1), n - 1, jnp.int32)
                for _ in range(_NCHAINS))
    m_c = jax.lax.cond(tie_conflict, run_tie, lambda: nm1)
    m16 = jnp.concatenate(m_c, axis=0).astype(jnp.int16)

    kept = gt | (eq & (idx16 <= m16))
    o_ref[0, 0] = jnp.where(kept, x, -jnp.inf)


def kernel(attn_weights, attention_mask):
    bsz, num_heads, seq_len, n = attn_weights.shape
    k = max(1, int(0.5 * seq_len))
    k = min(k, seq_len)

    rows_per_block = 512
    grid = (seq_len // rows_per_block, bsz * num_heads)

    out = pl.pallas_call(
        functools.partial(_topk_mask_kernel, k=k),
        grid=grid,
        in_specs=[
            pl.BlockSpec((1, 1, rows_per_block, n),
                         lambda rb, h: (0, h, rb, 0)),
            pl.BlockSpec((1, 1, rows_per_block, n),
                         lambda rb, h: (0, 0, rb, 0)),
        ],
        out_specs=pl.BlockSpec((1, 1, rows_per_block, n),
                               lambda rb, h: (0, h, rb, 0)),
        out_shape=jax.ShapeDtypeStruct(attn_weights.shape, jnp.float32),
        compiler_params=pltpu.CompilerParams(
            dimension_semantics=("parallel", "parallel"),
        ),
    )(attn_weights.reshape(1, bsz * num_heads, seq_len, n),
      attention_mask)
    return out.reshape(bsz, num_heads, seq_len, n)
